# initial kernel scaffold (unmeasured)
import jax
import jax.numpy as jnp
from jax import lax
from jax.experimental import pallas as pl
from jax.experimental.pallas import tpu as pltpu

N_DEV = 16


def kernel(x, w_mat):
    m, k_sh = x.shape
    _, n = w_mat.shape
    chunk = m // N_DEV

    def body(x_ref, w_ref, out_ref, p_ref, comm_ref, mx_ref,
             send_sems, recv_sems, credit_sem,
             mx_send_sems, mx_recv_sems, mx_credit_sem):
        my = lax.axis_index("i")
        left = jnp.mod(my - 1, N_DEV)
        right = jnp.mod(my + 1, N_DEV)

        p_ref[...] = jnp.dot(x_ref[...], w_ref[...],
                             preferred_element_type=jnp.float32)

        c0 = jnp.mod(my - 1, N_DEV)
        comm_ref[0, :, :] = p_ref[pl.ds(c0 * chunk, chunk), :]

        for s in range(N_DEV - 1):
            send_slot = s % 2
            recv_slot = (s + 1) % 2
            if s >= 1:
                pl.semaphore_wait(credit_sem, 1)
            rdma = pltpu.make_async_remote_copy(
                src_ref=comm_ref.at[send_slot],
                dst_ref=comm_ref.at[recv_slot],
                send_sem=send_sems.at[send_slot],
                recv_sem=recv_sems.at[recv_slot],
                device_id=(right,),
                device_id_type=pl.DeviceIdType.MESH,
            )
            rdma.start()
            rdma.wait()
            if s < N_DEV - 2:
                pl.semaphore_signal(credit_sem, inc=1, device_id=(left,),
                                    device_id_type=pl.DeviceIdType.MESH)
            c = jnp.mod(my - 2 - s, N_DEV)
            comm_ref[recv_slot, :, :] = (
                comm_ref[recv_slot, :, :] + p_ref[pl.ds(c * chunk, chunk), :]
            )

        y = jnp.maximum(comm_ref[1, :, :], 0.0)
        m_own = jnp.max(y)

        mx_ref[0, :, :] = jnp.broadcast_to(m_own, (8, 128))
        for s in range(N_DEV - 1):
            ss = s % 2
            rs = (s + 1) % 2
            if s >= 1:
                pl.semaphore_wait(mx_credit_sem, 1)
            rdma = pltpu.make_async_remote_copy(
                src_ref=mx_ref.at[ss],
                dst_ref=mx_ref.at[rs],
                send_sem=mx_send_sems.at[ss],
                recv_sem=mx_recv_sems.at[rs],
                device_id=(right,),
                device_id_type=pl.DeviceIdType.MESH,
            )
            rdma.start()
            rdma.wait()
            if s < N_DEV - 2:
                pl.semaphore_signal(mx_credit_sem, inc=1, device_id=(left,),
                                    device_id_type=pl.DeviceIdType.MESH)
            mx_ref[rs, :, :] = jnp.maximum(mx_ref[rs, :, :], m_own)

        gmax = mx_ref[1, 0, 0]
        scale = gmax / 127.0
        q = jnp.clip(jnp.round(y / scale), -127.0, 127.0)
        out_ref[...] = q * scale

    return pl.pallas_call(
        body,
        out_shape=jax.ShapeDtypeStruct((chunk, n), jnp.float32),
        in_specs=[
            pl.BlockSpec(memory_space=pltpu.VMEM),
            pl.BlockSpec(memory_space=pltpu.VMEM),
        ],
        out_specs=pl.BlockSpec(memory_space=pltpu.VMEM),
        scratch_shapes=[
            pltpu.VMEM((m, n), jnp.float32),
            pltpu.VMEM((2, chunk, n), jnp.float32),
            pltpu.VMEM((2, 8, 128), jnp.float32),
            pltpu.SemaphoreType.DMA((2,)),
            pltpu.SemaphoreType.DMA((2,)),
            pltpu.SemaphoreType.REGULAR,
            pltpu.SemaphoreType.DMA((2,)),
            pltpu.SemaphoreType.DMA((2,)),
            pltpu.SemaphoreType.REGULAR,
        ],
        compiler_params=pltpu.CompilerParams(collective_id=0),
    )(x, w_mat)


# baseline (device time: 509435 ns/iter reference)
import jax
import jax.numpy as jnp
from jax import lax
from jax.experimental import pallas as pl
from jax.experimental.pallas import tpu as pltpu

N_DEV = 16


def kernel(x, w_mat):
    m, k_sh = x.shape
    _, n = w_mat.shape
    chunk = m // N_DEV

    def body(x_ref, w_ref, out_ref, p_ref, comm_ref, mx_ref,
             send_sems, recv_sems, credit_sem,
             mx_send_sems, mx_recv_sems, mx_credit_sem):
        my = lax.axis_index("i")
        left = jnp.mod(my - 1, N_DEV)
        right = jnp.mod(my + 1, N_DEV)

        p_ref[...] = jnp.dot(x_ref[...], w_ref[...],
                             preferred_element_type=jnp.float32)

        c0 = jnp.mod(my - 1, N_DEV)
        comm_ref[0, :, :] = p_ref[pl.ds(c0 * chunk, chunk), :]

        for s in range(N_DEV - 1):
            send_slot = s % 2
            recv_slot = (s + 1) % 2
            if s >= 1:
                pl.semaphore_wait(credit_sem, 1)
            rdma = pltpu.make_async_remote_copy(
                src_ref=comm_ref.at[send_slot],
                dst_ref=comm_ref.at[recv_slot],
                send_sem=send_sems.at[send_slot],
                recv_sem=recv_sems.at[recv_slot],
                device_id=(right,),
                device_id_type=pl.DeviceIdType.MESH,
            )
            rdma.start()
            rdma.wait()
            if s < N_DEV - 2:
                pl.semaphore_signal(credit_sem, inc=1, device_id=(left,),
                                    device_id_type=pl.DeviceIdType.MESH)
            c = jnp.mod(my - 2 - s, N_DEV)
            comm_ref[recv_slot, :, :] = (
                comm_ref[recv_slot, :, :] + p_ref[pl.ds(c * chunk, chunk), :]
            )

        y = jnp.maximum(comm_ref[1, :, :], 0.0)
        m_own = jnp.max(y)

        mx_ref[0, :, :] = jnp.broadcast_to(m_own, (8, 128))
        for s in range(N_DEV - 1):
            ss = s % 2
            rs = (s + 1) % 2
            if s >= 1:
                pl.semaphore_wait(mx_credit_sem, 1)
            rdma = pltpu.make_async_remote_copy(
                src_ref=mx_ref.at[ss],
                dst_ref=mx_ref.at[rs],
                send_sem=mx_send_sems.at[ss],
                recv_sem=mx_recv_sems.at[rs],
                device_id=(right,),
                device_id_type=pl.DeviceIdType.MESH,
            )
            rdma.start()
            rdma.wait()
            if s < N_DEV - 2:
                pl.semaphore_signal(mx_credit_sem, inc=1, device_id=(left,),
                                    device_id_type=pl.DeviceIdType.MESH)
            mx_ref[rs, :, :] = jnp.maximum(mx_ref[rs, :, :], m_own)

        gmax = mx_ref[1, 0, 0]
        scale = gmax / 127.0
        q = jnp.clip(jnp.round(y / scale), -127.0, 127.0)
        out_ref[...] = q * scale

    return pl.pallas_call(
        body,
        out_shape=jax.ShapeDtypeStruct((chunk, n), jnp.float32),
        in_specs=[
            pl.BlockSpec(memory_space=pltpu.VMEM),
            pl.BlockSpec(memory_space=pltpu.VMEM),
        ],
        out_specs=pl.BlockSpec(memory_space=pltpu.VMEM),
        scratch_shapes=[
            pltpu.VMEM((m, n), jnp.float32),
            pltpu.VMEM((2, chunk, n), jnp.float32),
            pltpu.VMEM((2, 8, 128), jnp.float32),
            pltpu.SemaphoreType.DMA((2,)),
            pltpu.SemaphoreType.DMA((2,)),
            pltpu.SemaphoreType.REGULAR,
            pltpu.SemaphoreType.DMA((2,)),
            pltpu.SemaphoreType.DMA((2,)),
            pltpu.SemaphoreType.REGULAR,
        ],
        compiler_params=pltpu.CompilerParams(
            vmem_limit_bytes=100 * 1024 * 1024,
        ),
    )(x, w_mat)


# device time: 261997 ns/iter; 1.9444x vs baseline; 1.9444x over previous
import jax
import jax.numpy as jnp
from jax import lax
from jax.experimental import pallas as pl
from jax.experimental.pallas import tpu as pltpu

N_DEV = 16
MX_HOPS = 8


def kernel(x, w_mat):
    m, k_sh = x.shape
    _, n = w_mat.shape
    chunk = m // N_DEV
    nh = n // 2

    def body(x_ref, w_ref, out_ref,
             comm_cw, comm_ccw, t_cw, t_ccw, mx_cw, mx_ccw,
             cw_send_sems, cw_recv_sems, ccw_send_sems, ccw_recv_sems,
             credit_cw, credit_ccw,
             mxcw_send_sems, mxcw_recv_sems, mxccw_send_sems, mxccw_recv_sems,
             mx_credit_cw, mx_credit_ccw):
        my = lax.axis_index("i")
        left = jnp.mod(my - 1, N_DEV)
        right = jnp.mod(my + 1, N_DEV)

        def partial(c, lo):
            return jnp.dot(
                x_ref[pl.ds(c * chunk, chunk), :],
                w_ref[:, lo:lo + nh],
                preferred_element_type=jnp.float32,
            )

        comm_cw[0, :, :] = partial(jnp.mod(my - 1, N_DEV), 0)
        comm_ccw[0, :, :] = partial(jnp.mod(my + 1, N_DEV), nh)

        for s in range(N_DEV - 1):
            ss = s % 2
            rs = (s + 1) % 2
            if s >= 1:
                pl.semaphore_wait(credit_cw, 1)
                pl.semaphore_wait(credit_ccw, 1)
            rdma_cw = pltpu.make_async_remote_copy(
                src_ref=comm_cw.at[ss],
                dst_ref=comm_cw.at[rs],
                send_sem=cw_send_sems.at[ss],
                recv_sem=cw_recv_sems.at[rs],
                device_id=(right,),
                device_id_type=pl.DeviceIdType.MESH,
            )
            rdma_ccw = pltpu.make_async_remote_copy(
                src_ref=comm_ccw.at[ss],
                dst_ref=comm_ccw.at[rs],
                send_sem=ccw_send_sems.at[ss],
                recv_sem=ccw_recv_sems.at[rs],
                device_id=(left,),
                device_id_type=pl.DeviceIdType.MESH,
            )
            rdma_cw.start()
            rdma_ccw.start()

            c_cw = jnp.mod(my - 2 - s, N_DEV)
            c_ccw = jnp.mod(my + 2 + s, N_DEV)
            t_cw[...] = partial(c_cw, 0)
            t_ccw[...] = partial(c_ccw, nh)

            rdma_cw.wait_send()
            rdma_ccw.wait_send()
            if s < N_DEV - 2:
                pl.semaphore_signal(credit_cw, inc=1, device_id=(left,),
                                    device_id_type=pl.DeviceIdType.MESH)
                pl.semaphore_signal(credit_ccw, inc=1, device_id=(right,),
                                    device_id_type=pl.DeviceIdType.MESH)
            rdma_cw.wait_recv()
            rdma_ccw.wait_recv()
            comm_cw[rs, :, :] = comm_cw[rs, :, :] + t_cw[...]
            comm_ccw[rs, :, :] = comm_ccw[rs, :, :] + t_ccw[...]

        ycw = jnp.maximum(comm_cw[1, :, :], 0.0)
        yccw = jnp.maximum(comm_ccw[1, :, :], 0.0)
        m_own = jnp.maximum(jnp.max(ycw), jnp.max(yccw))

        mx_cw[0, :, :] = jnp.broadcast_to(m_own, (8, 128))
        mx_ccw[0, :, :] = jnp.broadcast_to(m_own, (8, 128))
        for s in range(MX_HOPS):
            ss = s % 2
            rs = (s + 1) % 2
            if s >= 1:
                pl.semaphore_wait(mx_credit_cw, 1)
                pl.semaphore_wait(mx_credit_ccw, 1)
            mxr_cw = pltpu.make_async_remote_copy(
                src_ref=mx_cw.at[ss],
                dst_ref=mx_cw.at[rs],
                send_sem=mxcw_send_sems.at[ss],
                recv_sem=mxcw_recv_sems.at[rs],
                device_id=(right,),
                device_id_type=pl.DeviceIdType.MESH,
            )
            mxr_ccw = pltpu.make_async_remote_copy(
                src_ref=mx_ccw.at[ss],
                dst_ref=mx_ccw.at[rs],
                send_sem=mxccw_send_sems.at[ss],
                recv_sem=mxccw_recv_sems.at[rs],
                device_id=(left,),
                device_id_type=pl.DeviceIdType.MESH,
            )
            mxr_cw.start()
            mxr_ccw.start()
            mxr_cw.wait_send()
            mxr_ccw.wait_send()
            if s < MX_HOPS - 1:
                pl.semaphore_signal(mx_credit_cw, inc=1, device_id=(left,),
                                    device_id_type=pl.DeviceIdType.MESH)
                pl.semaphore_signal(mx_credit_ccw, inc=1, device_id=(right,),
                                    device_id_type=pl.DeviceIdType.MESH)
            mxr_cw.wait_recv()
            mxr_ccw.wait_recv()
            mx_cw[rs, :, :] = jnp.maximum(mx_cw[rs, :, :], m_own)
            mx_ccw[rs, :, :] = jnp.maximum(mx_ccw[rs, :, :], m_own)

        final = MX_HOPS % 2
        gmax = jnp.maximum(mx_cw[final, 0, 0], mx_ccw[final, 0, 0])
        scale = gmax / 127.0
        qcw = jnp.clip(jnp.round(ycw / scale), -127.0, 127.0)
        qccw = jnp.clip(jnp.round(yccw / scale), -127.0, 127.0)
        out_ref[:, 0:nh] = qcw * scale
        out_ref[:, nh:n] = qccw * scale

    return pl.pallas_call(
        body,
        out_shape=jax.ShapeDtypeStruct((chunk, n), jnp.float32),
        in_specs=[
            pl.BlockSpec(memory_space=pltpu.VMEM),
            pl.BlockSpec(memory_space=pltpu.VMEM),
        ],
        out_specs=pl.BlockSpec(memory_space=pltpu.VMEM),
        scratch_shapes=[
            pltpu.VMEM((2, chunk, nh), jnp.float32),
            pltpu.VMEM((2, chunk, nh), jnp.float32),
            pltpu.VMEM((chunk, nh), jnp.float32),
            pltpu.VMEM((chunk, nh), jnp.float32),
            pltpu.VMEM((2, 8, 128), jnp.float32),
            pltpu.VMEM((2, 8, 128), jnp.float32),
            pltpu.SemaphoreType.DMA((2,)),
            pltpu.SemaphoreType.DMA((2,)),
            pltpu.SemaphoreType.DMA((2,)),
            pltpu.SemaphoreType.DMA((2,)),
            pltpu.SemaphoreType.REGULAR,
            pltpu.SemaphoreType.REGULAR,
            pltpu.SemaphoreType.DMA((2,)),
            pltpu.SemaphoreType.DMA((2,)),
            pltpu.SemaphoreType.DMA((2,)),
            pltpu.SemaphoreType.DMA((2,)),
            pltpu.SemaphoreType.REGULAR,
            pltpu.SemaphoreType.REGULAR,
        ],
        compiler_params=pltpu.CompilerParams(
            vmem_limit_bytes=100 * 1024 * 1024,
        ),
    )(x, w_mat)


# device time: 193595 ns/iter; 2.6314x vs baseline; 1.3533x over previous
import jax
import jax.numpy as jnp
from jax import lax
from jax.experimental import pallas as pl
from jax.experimental.pallas import tpu as pltpu

N_DEV = 16


def kernel(x, w_mat):
    m, k_sh = x.shape
    _, n = w_mat.shape
    chunk = m // N_DEV
    q = n // 4

    def body(x_ref, w_ref, out_ref,
             comm0, comm1, comm2, comm3, t0, t1, gather,
             send0, recv0, send1, recv1, send2, recv2, send3, recv3,
             cred0, cred1, cred2, cred3,
             aa_send, aa_recv):
        my = lax.axis_index("i")
        left = jnp.mod(my - 1, N_DEV)
        right = jnp.mod(my + 1, N_DEV)

        def partial(c, lo):
            return jnp.dot(
                x_ref[pl.ds(c * chunk, chunk), :],
                w_ref[:, lo:lo + q],
                preferred_element_type=jnp.float32,
            )

        class Ring:
            def __init__(self, comm, send_sems, recv_sems, credit, lo, cw):
                self.comm = comm
                self.send_sems = send_sems
                self.recv_sems = recv_sems
                self.credit = credit
                self.lo = lo
                self.cw = cw
                self.target = right if cw else left
                self.credit_to = left if cw else right
                self.prev_send = None

            def seed_chunk(self):
                return jnp.mod(my - 1, N_DEV) if self.cw else jnp.mod(my + 1, N_DEV)

            def recv_chunk(self, s):
                return (jnp.mod(my - 2 - s, N_DEV) if self.cw
                        else jnp.mod(my + 2 + s, N_DEV))

            def start_send(self, u):
                d = pltpu.make_async_remote_copy(
                    src_ref=self.comm.at[u % 2],
                    dst_ref=self.comm.at[(u + 1) % 2],
                    send_sem=self.send_sems.at[u % 2],
                    recv_sem=self.recv_sems.at[(u + 1) % 2],
                    device_id=(self.target,),
                    device_id_type=pl.DeviceIdType.MESH,
                )
                d.start()
                self.prev_send = d

            def wait_recv(self, s):
                rs = (s + 1) % 2
                d = pltpu.make_async_remote_copy(
                    src_ref=self.comm.at[rs],
                    dst_ref=self.comm.at[rs],
                    send_sem=self.send_sems.at[rs],
                    recv_sem=self.recv_sems.at[rs],
                    device_id=(self.target,),
                    device_id_type=pl.DeviceIdType.MESH,
                )
                d.wait_recv()

        grp_a = [Ring(comm0, send0, recv0, cred0, 0 * q, True),
                 Ring(comm2, send2, recv2, cred2, 2 * q, False)]
        grp_b = [Ring(comm1, send1, recv1, cred1, 1 * q, True),
                 Ring(comm3, send3, recv3, cred3, 3 * q, False)]
        groups = [grp_a, grp_b]
        tmps = [t0, t1]

        for grp in groups:
            for r in grp:
                r.comm[0, :, :] = partial(r.seed_chunk(), r.lo)
            for r in grp:
                r.start_send(0)

        for s in range(N_DEV - 1):
            rs = (s + 1) % 2
            for grp in groups:
                for r, t in zip(grp, tmps):
                    t[...] = partial(r.recv_chunk(s), r.lo)
                for r in grp:
                    r.wait_recv(s)
                    r.prev_send.wait_send()
                if s < N_DEV - 2:
                    for r in grp:
                        pl.semaphore_signal(
                            r.credit, inc=1, device_id=(r.credit_to,),
                            device_id_type=pl.DeviceIdType.MESH)
                for r, t in zip(grp, tmps):
                    r.comm[rs, :, :] = r.comm[rs, :, :] + t[...]
                if s < N_DEV - 2:
                    for r in grp:
                        pl.semaphore_wait(r.credit, 1)
                        r.start_send(s + 1)

        ys = [jnp.maximum(r.comm[1, :, :], 0.0) for r in grp_a + grp_b]
        m_own = jnp.max(jnp.stack([jnp.max(y) for y in ys]))

        gather[pl.ds(my, 1)] = jnp.broadcast_to(m_own, (1, 8, 128))
        sends = []
        for k in range(1, N_DEV):
            tgt = jnp.mod(my + k, N_DEV)
            d = pltpu.make_async_remote_copy(
                src_ref=gather.at[my],
                dst_ref=gather.at[my],
                send_sem=aa_send.at[tgt],
                recv_sem=aa_recv.at[my],
                device_id=(tgt,),
                device_id_type=pl.DeviceIdType.MESH,
            )
            d.start()
            sends.append(d)
        for k in range(1, N_DEV):
            src = jnp.mod(my + k, N_DEV)
            d = pltpu.make_async_remote_copy(
                src_ref=gather.at[src],
                dst_ref=gather.at[src],
                send_sem=aa_send.at[src],
                recv_sem=aa_recv.at[src],
                device_id=(src,),
                device_id_type=pl.DeviceIdType.MESH,
            )
            d.wait_recv()
        for d in sends:
            d.wait_send()

        gmax = jnp.max(gather[...])
        scale = gmax / 127.0
        for r, y in zip(grp_a + grp_b, ys):
            qv = jnp.clip(jnp.round(y / scale), -127.0, 127.0)
            out_ref[:, r.lo:r.lo + q] = qv * scale

    return pl.pallas_call(
        body,
        out_shape=jax.ShapeDtypeStruct((chunk, n), jnp.float32),
        in_specs=[
            pl.BlockSpec(memory_space=pltpu.VMEM),
            pl.BlockSpec(memory_space=pltpu.VMEM),
        ],
        out_specs=pl.BlockSpec(memory_space=pltpu.VMEM),
        scratch_shapes=[
            pltpu.VMEM((2, chunk, q), jnp.float32),
            pltpu.VMEM((2, chunk, q), jnp.float32),
            pltpu.VMEM((2, chunk, q), jnp.float32),
            pltpu.VMEM((2, chunk, q), jnp.float32),
            pltpu.VMEM((chunk, q), jnp.float32),
            pltpu.VMEM((chunk, q), jnp.float32),
            pltpu.VMEM((N_DEV, 8, 128), jnp.float32),
            pltpu.SemaphoreType.DMA((2,)), pltpu.SemaphoreType.DMA((2,)),
            pltpu.SemaphoreType.DMA((2,)), pltpu.SemaphoreType.DMA((2,)),
            pltpu.SemaphoreType.DMA((2,)), pltpu.SemaphoreType.DMA((2,)),
            pltpu.SemaphoreType.DMA((2,)), pltpu.SemaphoreType.DMA((2,)),
            pltpu.SemaphoreType.REGULAR, pltpu.SemaphoreType.REGULAR,
            pltpu.SemaphoreType.REGULAR, pltpu.SemaphoreType.REGULAR,
            pltpu.SemaphoreType.DMA((N_DEV,)),
            pltpu.SemaphoreType.DMA((N_DEV,)),
        ],
        compiler_params=pltpu.CompilerParams(
            vmem_limit_bytes=100 * 1024 * 1024,
        ),
    )(x, w_mat)


# device time: 190033 ns/iter; 2.6808x vs baseline; 1.0187x over previous
import jax
import jax.numpy as jnp
from jax import lax
from jax.experimental import pallas as pl
from jax.experimental.pallas import tpu as pltpu

N_DEV = 16
D = 4


def kernel(x, w_mat):
    m, k_sh = x.shape
    _, n = w_mat.shape
    chunk = m // N_DEV
    q = n // 4

    def body(x_ref, w_ref, out_ref,
             comm0, comm1, comm2, comm3, t0, t1, gather,
             send0, recv0, send1, recv1, send2, recv2, send3, recv3,
             cred0, cred1, cred2, cred3,
             aa_send, aa_recv):
        my = lax.axis_index("i")
        left = jnp.mod(my - 1, N_DEV)
        right = jnp.mod(my + 1, N_DEV)

        barrier_sem = pltpu.get_barrier_semaphore()
        for nbr in (left, right):
            pl.semaphore_signal(barrier_sem, inc=1, device_id=(nbr,),
                                device_id_type=pl.DeviceIdType.MESH)
        pl.semaphore_wait(barrier_sem, 2)

        def partial(c, lo):
            return jnp.dot(
                x_ref[pl.ds(c * chunk, chunk), :],
                w_ref[:, lo:lo + q],
                preferred_element_type=jnp.float32,
            )

        class Ring:
            def __init__(self, comm, send_sems, recv_sems, credit, lo, cw):
                self.comm = comm
                self.send_sems = send_sems
                self.recv_sems = recv_sems
                self.credit = credit
                self.lo = lo
                self.cw = cw
                self.target = right if cw else left
                self.credit_to = left if cw else right
                self.prev_send = None

            def seed_chunk(self):
                return jnp.mod(my - 1, N_DEV) if self.cw else jnp.mod(my + 1, N_DEV)

            def recv_chunk(self, s):
                return (jnp.mod(my - 2 - s, N_DEV) if self.cw
                        else jnp.mod(my + 2 + s, N_DEV))

            def start_send(self, u):
                d = pltpu.make_async_remote_copy(
                    src_ref=self.comm.at[u % D],
                    dst_ref=self.comm.at[(u + 1) % D],
                    send_sem=self.send_sems.at[u % D],
                    recv_sem=self.recv_sems.at[(u + 1) % D],
                    device_id=(self.target,),
                    device_id_type=pl.DeviceIdType.MESH,
                )
                d.start()
                self.prev_send = d

            def wait_recv(self, s):
                rs = (s + 1) % D
                d = pltpu.make_async_remote_copy(
                    src_ref=self.comm.at[rs],
                    dst_ref=self.comm.at[rs],
                    send_sem=self.send_sems.at[rs],
                    recv_sem=self.recv_sems.at[rs],
                    device_id=(self.target,),
                    device_id_type=pl.DeviceIdType.MESH,
                )
                d.wait_recv()

        grp_a = [Ring(comm0, send0, recv0, cred0, 0 * q, True),
                 Ring(comm2, send2, recv2, cred2, 2 * q, False)]
        grp_b = [Ring(comm1, send1, recv1, cred1, 1 * q, True),
                 Ring(comm3, send3, recv3, cred3, 3 * q, False)]
        groups = [grp_a, grp_b]
        tmps = [t0, t1]

        for grp in groups:
            for r in grp:
                r.comm[0, :, :] = partial(r.seed_chunk(), r.lo)
            for r in grp:
                r.start_send(0)

        for s in range(N_DEV - 1):
            rs = (s + 1) % D
            for grp in groups:
                for r, t in zip(grp, tmps):
                    t[...] = partial(r.recv_chunk(s), r.lo)
                for r in grp:
                    r.wait_recv(s)
                    r.prev_send.wait_send()
                if s <= (N_DEV - 2) - (D - 1):
                    for r in grp:
                        pl.semaphore_signal(
                            r.credit, inc=1, device_id=(r.credit_to,),
                            device_id_type=pl.DeviceIdType.MESH)
                for r, t in zip(grp, tmps):
                    r.comm[rs, :, :] = r.comm[rs, :, :] + t[...]
                if s < N_DEV - 2:
                    for r in grp:
                        if s + 1 >= D - 1:
                            pl.semaphore_wait(r.credit, 1)
                        r.start_send(s + 1)

        fin = (N_DEV - 1) % D
        ys = [jnp.maximum(r.comm[fin, :, :], 0.0) for r in grp_a + grp_b]
        m_own = jnp.max(jnp.stack([jnp.max(y) for y in ys]))

        gather[pl.ds(my, 1)] = jnp.broadcast_to(m_own, (1, 8, 128))
        sends = []
        for k in range(1, N_DEV):
            tgt = jnp.mod(my + k, N_DEV)
            d = pltpu.make_async_remote_copy(
                src_ref=gather.at[my],
                dst_ref=gather.at[my],
                send_sem=aa_send.at[tgt],
                recv_sem=aa_recv.at[my],
                device_id=(tgt,),
                device_id_type=pl.DeviceIdType.MESH,
            )
            d.start()
            sends.append(d)
        for k in range(1, N_DEV):
            src = jnp.mod(my + k, N_DEV)
            d = pltpu.make_async_remote_copy(
                src_ref=gather.at[src],
                dst_ref=gather.at[src],
                send_sem=aa_send.at[src],
                recv_sem=aa_recv.at[src],
                device_id=(src,),
                device_id_type=pl.DeviceIdType.MESH,
            )
            d.wait_recv()
        for d in sends:
            d.wait_send()

        gmax = jnp.max(gather[...])
        scale = gmax / 127.0
        for r, y in zip(grp_a + grp_b, ys):
            qv = jnp.clip(jnp.round(y / scale), -127.0, 127.0)
            out_ref[:, r.lo:r.lo + q] = qv * scale

    return pl.pallas_call(
        body,
        out_shape=jax.ShapeDtypeStruct((chunk, n), jnp.float32),
        in_specs=[
            pl.BlockSpec(memory_space=pltpu.VMEM),
            pl.BlockSpec(memory_space=pltpu.VMEM),
        ],
        out_specs=pl.BlockSpec(memory_space=pltpu.VMEM),
        scratch_shapes=[
            pltpu.VMEM((D, chunk, q), jnp.float32),
            pltpu.VMEM((D, chunk, q), jnp.float32),
            pltpu.VMEM((D, chunk, q), jnp.float32),
            pltpu.VMEM((D, chunk, q), jnp.float32),
            pltpu.VMEM((chunk, q), jnp.float32),
            pltpu.VMEM((chunk, q), jnp.float32),
            pltpu.VMEM((N_DEV, 8, 128), jnp.float32),
            pltpu.SemaphoreType.DMA((D,)), pltpu.SemaphoreType.DMA((D,)),
            pltpu.SemaphoreType.DMA((D,)), pltpu.SemaphoreType.DMA((D,)),
            pltpu.SemaphoreType.DMA((D,)), pltpu.SemaphoreType.DMA((D,)),
            pltpu.SemaphoreType.DMA((D,)), pltpu.SemaphoreType.DMA((D,)),
            pltpu.SemaphoreType.REGULAR, pltpu.SemaphoreType.REGULAR,
            pltpu.SemaphoreType.REGULAR, pltpu.SemaphoreType.REGULAR,
            pltpu.SemaphoreType.DMA((N_DEV,)),
            pltpu.SemaphoreType.DMA((N_DEV,)),
        ],
        compiler_params=pltpu.CompilerParams(
            collective_id=0,
            vmem_limit_bytes=100 * 1024 * 1024,
        ),
    )(x, w_mat)
